# Initial kernel scaffold; baseline (speedup 1.0000x reference)
import jax
import jax.numpy as jnp
from jax.experimental import pallas as pl


def kernel(X, edge_index, Wz, bz, Wr, br, Wh, bh, LzW, Lzb, LrW, Lrb, LhW, Lhb):
    raise NotImplementedError("write your pallas kernel here")


# speedup vs baseline: 8.0266x; 8.0266x over previous
import functools

import jax
import jax.numpy as jnp
from jax import lax
from jax.experimental import pallas as pl
from jax.experimental.pallas import tpu as pltpu
from jax.experimental.pallas import tpu_sc as plsc

N = 10000
E = 160000
C = 256
HALF = 128
NH = 5000
NC = 2
NS = 16
CHUNK = 128
NCH_H = 40
NCH_S = 80
EP = NC * NS * NCH_H * CHUNK
NBUF = 4
_B = 1000

NP_S = 5120
DUMP_S = NH
ZR_S = NP_S // NS
RO_S = 312
_ZCH_S = [(0, 128), (128, 128), (256, ZR_S - 256)]
_RCH_S = [(0, 128), (128, 128), (256, RO_S - 256)]
_TAIL_S = NH - NS * RO_S


def _mesh():
    return plsc.VectorSubcoreMesh(core_axis_name="c", subcore_axis_name="s")


@functools.partial(
    pl.kernel,
    out_type=jax.ShapeDtypeStruct((NC, NH, HALF), jnp.float32),
    mesh=_mesh(),
    scratch_types=[
        pltpu.VMEM((NCH_S, CHUNK), jnp.int32),
        pltpu.VMEM((CHUNK, HALF), jnp.float32),
        pltpu.VMEM_SHARED((NP_S, HALF), jnp.float32),
    ],
)
def _hist(dstc, ones, z128, out, dst_v, ones_v, cnt_s):
    c = lax.axis_index("c")
    s = lax.axis_index("s")
    pltpu.sync_copy(dstc.at[c, s], dst_v)
    pltpu.sync_copy(z128, ones_v)
    for off, sz in _ZCH_S:
        pltpu.sync_copy(ones_v.at[pl.ds(0, sz)],
                        cnt_s.at[pl.ds(s * ZR_S + off, sz)])
    pltpu.sync_copy(ones, ones_v)
    plsc.subcore_barrier()

    def body(j, carry):
        pltpu.sync_copy(ones_v, cnt_s.at[dst_v.at[j]], add=True)
        return carry

    lax.fori_loop(0, NCH_S, body, 0)
    plsc.subcore_barrier()
    for off, sz in _RCH_S:
        pltpu.sync_copy(cnt_s.at[pl.ds(s * RO_S + off, sz)], ones_v.at[pl.ds(0, sz)])
        pltpu.sync_copy(ones_v.at[pl.ds(0, sz)],
                        out.at[c, pl.ds(s * RO_S + off, sz)])

    @pl.when(s == 0)
    def _tail():
        pltpu.sync_copy(cnt_s.at[pl.ds(NS * RO_S, _TAIL_S)],
                        ones_v.at[pl.ds(0, _TAIL_S)])
        pltpu.sync_copy(ones_v.at[pl.ds(0, _TAIL_S)],
                        out.at[c, pl.ds(NS * RO_S, _TAIL_S)])


@functools.partial(
    pl.kernel,
    out_type=jax.ShapeDtypeStruct((NC, NH, HALF), jnp.float32),
    mesh=_mesh(),
    scratch_types=[
        pltpu.VMEM((NCH_S, CHUNK), jnp.int32),
        pltpu.VMEM((NCH_S, CHUNK), jnp.int32),
        pltpu.VMEM((NBUF, CHUNK, HALF), jnp.float32),
        pltpu.VMEM_SHARED((NP_S, HALF), jnp.float32),
        pltpu.SemaphoreType.DMA,
        pltpu.SemaphoreType.DMA,
        pltpu.SemaphoreType.DMA,
        pltpu.SemaphoreType.DMA,
    ],
)
def _scatter(srcr, dstc, y2, z128, out, src_v, dst_v, rows_v, t_s, s0, s1, s2, s3):
    sems = [s0, s1, s2, s3]
    c = lax.axis_index("c")
    s = lax.axis_index("s")
    pltpu.sync_copy(srcr.at[s], src_v)
    pltpu.sync_copy(dstc.at[c, s], dst_v)
    pltpu.sync_copy(z128, rows_v.at[0])
    for off, sz in _ZCH_S:
        pltpu.sync_copy(rows_v.at[0, pl.ds(0, sz)],
                        t_s.at[pl.ds(s * ZR_S + off, sz)])
    plsc.subcore_barrier()

    for b in range(NBUF):
        pltpu.async_copy(y2.at[src_v.at[b]], rows_v.at[b], sems[b])

    def body(g, carry):
        for b in range(NBUF):
            j = g * NBUF + b
            pltpu.make_async_copy(y2.at[src_v.at[j]], rows_v.at[b], sems[b]).wait()
            pltpu.sync_copy(rows_v.at[b], t_s.at[dst_v.at[j]], add=True)

            @pl.when(j + NBUF < NCH_S)
            def _start():
                pltpu.async_copy(y2.at[src_v.at[j + NBUF]], rows_v.at[b], sems[b])

        return carry

    lax.fori_loop(0, NCH_S // NBUF, body, 0)
    plsc.subcore_barrier()
    for off, sz in _RCH_S:
        pltpu.sync_copy(t_s.at[pl.ds(s * RO_S + off, sz)], rows_v.at[0, pl.ds(0, sz)])
        pltpu.sync_copy(rows_v.at[0, pl.ds(0, sz)],
                        out.at[c, pl.ds(s * RO_S + off, sz)])

    @pl.when(s == 0)
    def _tail():
        pltpu.sync_copy(t_s.at[pl.ds(NS * RO_S, _TAIL_S)],
                        rows_v.at[0, pl.ds(0, _TAIL_S)])
        pltpu.sync_copy(rows_v.at[0, pl.ds(0, _TAIL_S)],
                        out.at[c, pl.ds(NS * RO_S, _TAIL_S)])


def _scale_body(cnt_ref, x_ref, y2_ref, dis_ref):
    deg = cnt_ref[:, 0:1] + 1.0
    d = lax.rsqrt(deg)
    y = x_ref[...] * d
    y2_ref[0] = y[:, :HALF]
    y2_ref[1] = y[:, HALF:]
    dis_ref[...] = d


def _scale(cnt, x):
    nb = N // _B
    return pl.pallas_call(
        _scale_body,
        grid=(nb,),
        in_specs=[
            pl.BlockSpec((_B, HALF), lambda i: (i, 0)),
            pl.BlockSpec((_B, C), lambda i: (i, 0)),
        ],
        out_specs=[
            pl.BlockSpec((2, _B, HALF), lambda i: (0, i, 0)),
            pl.BlockSpec((_B, 1), lambda i: (i, 0)),
        ],
        out_shape=[
            jax.ShapeDtypeStruct((2, N, HALF), jnp.float32),
            jax.ShapeDtypeStruct((N, 1), jnp.float32),
        ],
    )(cnt, x)


def _fold_body(wz_ref, lz_ref, bz_ref, lzb_ref, wh_ref, lh_ref, bh_ref, lhb_ref,
               mz_ref, cz_ref, mh_ref, ch_ref):
    mz_ref[...] = jnp.dot(wz_ref[...], lz_ref[...], preferred_element_type=jnp.float32)
    cz_ref[...] = jnp.dot(bz_ref[...], lz_ref[...], preferred_element_type=jnp.float32) + lzb_ref[...]
    mh_ref[...] = jnp.dot(wh_ref[...], lh_ref[...], preferred_element_type=jnp.float32)
    ch_ref[...] = jnp.dot(bh_ref[...], lh_ref[...], preferred_element_type=jnp.float32) + lhb_ref[...]


def _fold(wz, lz, bz, lzb, wh, lh, bh, lhb):
    return pl.pallas_call(
        _fold_body,
        out_shape=[
            jax.ShapeDtypeStruct((C, C), jnp.float32),
            jax.ShapeDtypeStruct((1, C), jnp.float32),
            jax.ShapeDtypeStruct((C, C), jnp.float32),
            jax.ShapeDtypeStruct((1, C), jnp.float32),
        ],
    )(wz, lz, bz, lzb, wh, lh, bh, lhb)


def _final_body(tl_ref, tr_ref, yl_ref, yr_ref, dis_ref,
                mz_ref, cz_ref, mh_ref, ch_ref, out_ref):
    d = dis_ref[...]
    sl = (tl_ref[...] + yl_ref[0]) * d
    sr = (tr_ref[...] + yr_ref[0]) * d
    mz = mz_ref[...]
    mh = mh_ref[...]
    az = (jnp.dot(sl, mz[:HALF], preferred_element_type=jnp.float32)
          + jnp.dot(sr, mz[HALF:], preferred_element_type=jnp.float32)
          + cz_ref[...])
    ah = (jnp.dot(sl, mh[:HALF], preferred_element_type=jnp.float32)
          + jnp.dot(sr, mh[HALF:], preferred_element_type=jnp.float32)
          + ch_ref[...])
    out_ref[...] = (1.0 - jax.nn.sigmoid(az)) * jnp.tanh(ah)


def _final(tl, tr, y2, dis, mz, cz, mh, ch):
    nb = N // _B
    return pl.pallas_call(
        _final_body,
        grid=(nb,),
        in_specs=[
            pl.BlockSpec((_B, HALF), lambda i: (i, 0)),
            pl.BlockSpec((_B, HALF), lambda i: (i, 0)),
            pl.BlockSpec((1, _B, HALF), lambda i: (0, i, 0)),
            pl.BlockSpec((1, _B, HALF), lambda i: (1, i, 0)),
            pl.BlockSpec((_B, 1), lambda i: (i, 0)),
            pl.BlockSpec((C, C), lambda i: (0, 0)),
            pl.BlockSpec((1, C), lambda i: (0, 0)),
            pl.BlockSpec((C, C), lambda i: (0, 0)),
            pl.BlockSpec((1, C), lambda i: (0, 0)),
        ],
        out_specs=pl.BlockSpec((_B, C), lambda i: (i, 0)),
        out_shape=jax.ShapeDtypeStruct((N, C), jnp.float32),
    )(tl, tr, y2, y2, dis, mz, cz, mh, ch)


def kernel(X, edge_index, Wz, bz, Wr, br, Wh, bh, LzW, Lzb, LrW, Lrb, LhW, Lhb):
    src = edge_index[0].astype(jnp.int32)
    dst = edge_index[1].astype(jnp.int32)
    pad = EP - E
    srcp = jnp.concatenate([src, jnp.zeros((pad,), jnp.int32)])
    dstp = jnp.concatenate([dst, jnp.full((pad,), N, jnp.int32)])

    d0 = jnp.where(dstp < NH, dstp, DUMP_S)
    d1m = dstp - NH
    d1 = jnp.where((d1m >= 0) & (d1m < NH), d1m, DUMP_S)
    dstc = jnp.stack([d0, d1]).reshape(NC, NS, NCH_S, CHUNK)
    srcr0 = srcp.reshape(NS, NCH_S, CHUNK)
    srcr1 = (srcp + N).reshape(NS, NCH_S, CHUNK)

    ones128 = jnp.ones((CHUNK, HALF), jnp.float32)
    z128 = jnp.zeros((CHUNK, HALF), jnp.float32)

    cnt = _hist(dstc, ones128, z128).reshape(N, HALF)
    y2, dis = _scale(cnt, X)
    y2f = y2.reshape(2 * N, HALF)
    mz, cz, mh, ch = _fold(
        Wz, LzW[:C], bz.reshape(1, C), Lzb.reshape(1, C),
        Wh, LhW[:C], bh.reshape(1, C), Lhb.reshape(1, C),
    )
    tl = _scatter(srcr0, dstc, y2f, z128).reshape(N, HALF)
    tr = _scatter(srcr1, dstc, y2f, z128).reshape(N, HALF)
    return _final(tl, tr, y2, dis, mz, cz, mh, ch)
